# baseline (speedup 1.0000x reference)
import functools

import jax
import jax.numpy as jnp
from jax import lax
from jax.experimental import pallas as pl
from jax.experimental.pallas import tpu as pltpu
from jax.experimental.pallas import tpu_sc as plsc

N = 10000
E = 320000
D = 128
NC = 2
NS = 16
NW = NC * NS
NPAD = 10240
RPS = NPAD // NS
CHUNK = 64
IBLK = 16
NBUF = 4
KTOT = 320
TOTC = NS * KTOT
EPAD = TOTC * CHUNK
NCHUNK = TOTC // NW
K0 = 320
K1 = 0
assert K0 + K1 == KTOT and K0 % IBLK == 0 and K1 % IBLK == 0
DEGW = 128

_HIGH = lax.Precision.HIGHEST


def _sc_mesh():
    return plsc.VectorSubcoreMesh(core_axis_name="c", subcore_axis_name="s")


def _sc_degree(dst_idx):
    @functools.partial(
        pl.kernel,
        out_type=jax.ShapeDtypeStruct((NC, NPAD, DEGW), jnp.float32),
        mesh=_sc_mesh(),
        scratch_types=[
            pltpu.VMEM((NCHUNK, CHUNK), jnp.int32),
            pltpu.VMEM((CHUNK, DEGW), jnp.float32),
            pltpu.VMEM((CHUNK, DEGW), jnp.float32),
            pltpu.VMEM_SHARED((NPAD, DEGW), jnp.float32),
            pltpu.SemaphoreType.DMA,
        ],
    )
    def k(d_hbm, out_hbm, didx, zbuf, ones, acc, sem):
        c = lax.axis_index("c")
        s = lax.axis_index("s")
        w = c * NS + s
        pltpu.sync_copy(d_hbm.at[pl.ds(w * NCHUNK, NCHUNK)], didx)

        @pl.loop(0, CHUNK)
        def _(i):
            zbuf.at[i, pl.ds(0, DEGW)][...] = jnp.zeros((DEGW,), jnp.float32)
            ones.at[i, pl.ds(0, DEGW)][...] = jnp.full((DEGW,), 1.0, jnp.float32)

        @pl.loop(0, RPS, step=CHUNK)
        def _(r):
            pltpu.sync_copy(zbuf, acc.at[pl.ds(s * RPS + r, CHUNK)])

        plsc.subcore_barrier()

        @pl.loop(0, NCHUNK, step=8)
        def _(j):
            for t in range(8):
                pltpu.async_copy(ones, acc.at[didx.at[j + t]], sem, add=True)
            for t in range(8):
                pltpu.make_async_copy(ones, acc.at[didx.at[j + t]], sem).wait()

        plsc.subcore_barrier()
        pltpu.sync_copy(acc.at[pl.ds(s * RPS, RPS)],
                        out_hbm.at[c, pl.ds(s * RPS, RPS)])

    return k(dst_idx)


def _sc_propagate(u_pad, src_idx, dst_idx):
    @functools.partial(
        pl.kernel,
        out_type=jax.ShapeDtypeStruct((NC, NPAD, D), jnp.float32),
        mesh=_sc_mesh(),
        scratch_types=[
            pltpu.VMEM((IBLK, CHUNK), jnp.int32),
            pltpu.VMEM((IBLK, CHUNK), jnp.int32),
        ] + [pltpu.VMEM((CHUNK, D), jnp.float32) for _ in range(NBUF)] + [
            pltpu.VMEM_SHARED((NPAD, D), jnp.float32),
        ] + [pltpu.SemaphoreType.DMA for _ in range(NBUF)],
    )
    def k(u_hbm, s_hbm, d_hbm, out_hbm, sidx, didx, *rest):
        rows = rest[:NBUF]
        acc = rest[NBUF]
        sems = rest[NBUF + 1:]
        c = lax.axis_index("c")
        s = lax.axis_index("s")
        u_mine = u_hbm.at[c]
        base = jnp.where(c == 0, s * K0, NS * K0 + s * K1)
        count = jnp.where(c == 0, K0, K1)

        @pl.loop(0, CHUNK)
        def _(i):
            @pl.loop(0, D, step=16)
            def _(j):
                rows[0].at[i, pl.ds(j, 16)][...] = jnp.zeros((16,), jnp.float32)

        @pl.loop(0, RPS, step=CHUNK)
        def _(r):
            pltpu.sync_copy(rows[0], acc.at[pl.ds(s * RPS + r, CHUNK)])

        plsc.subcore_barrier()

        def gather(j, q):
            return pltpu.async_copy(u_mine.at[sidx.at[j]], rows[q], sems[q])

        def gather_wait(j, q):
            pltpu.make_async_copy(u_mine.at[sidx.at[j]], rows[q], sems[q]).wait()

        @pl.loop(0, count, step=IBLK)
        def _(b):
            pltpu.sync_copy(s_hbm.at[pl.ds(base + b, IBLK)], sidx)
            pltpu.sync_copy(d_hbm.at[pl.ds(base + b, IBLK)], didx)
            for q in range(NBUF):
                gather(q, q)

            @pl.loop(0, IBLK, step=NBUF)
            def _(j):
                for q in range(NBUF):
                    gather_wait(j + q, q)
                    pltpu.sync_copy(rows[q], acc.at[didx.at[j + q]], add=True)

                    @pl.when(j + q + NBUF < IBLK)
                    def _():
                        gather(j + q + NBUF, q)

        plsc.subcore_barrier()
        pltpu.sync_copy(acc.at[pl.ds(s * RPS, RPS)],
                        out_hbm.at[c, pl.ds(s * RPS, RPS)])

    return k(u_pad, src_idx, dst_idx)


_BLK = 512
_GRID = (NPAD // _BLK,)


def _tc_first(x, w, degp):
    def body(x_ref, w_ref, deg_ref, u_ref, dinv_ref):
        deg = deg_ref[0][:, 0:1] + deg_ref[1][:, 0:1] + 1.0
        dinv = lax.rsqrt(deg)
        xw = jnp.dot(x_ref[...], w_ref[...], precision=_HIGH,
                     preferred_element_type=jnp.float32)
        u = xw * dinv
        u_ref[0] = u
        u_ref[1] = u
        dinv_ref[...] = jnp.broadcast_to(dinv, (_BLK, DEGW))

    return pl.pallas_call(
        body,
        grid=_GRID,
        in_specs=[pl.BlockSpec((_BLK, D), lambda i: (i, 0)),
                  pl.BlockSpec((D, D), lambda i: (0, 0)),
                  pl.BlockSpec((NC, _BLK, DEGW), lambda i: (0, i, 0))],
        out_specs=[pl.BlockSpec((NC, _BLK, D), lambda i: (0, i, 0)),
                   pl.BlockSpec((_BLK, DEGW), lambda i: (i, 0))],
        out_shape=[jax.ShapeDtypeStruct((NC, NPAD, D), jnp.float32),
                   jax.ShapeDtypeStruct((NPAD, DEGW), jnp.float32)],
    )(x, w, degp)


def _tc_mid(sp, u1, dinv16, w2, b1):
    def body(sp_ref, u_ref, dinv_ref, w_ref, b_ref, o_ref):
        dinv = dinv_ref[:, 0:1]
        pre = dinv * (sp_ref[0] + sp_ref[1] + u_ref[0]) + b_ref[...]
        h = jnp.maximum(pre, 0.0)
        u2 = jnp.dot(h, w_ref[...], precision=_HIGH,
                     preferred_element_type=jnp.float32) * dinv
        o_ref[0] = u2
        o_ref[1] = u2

    return pl.pallas_call(
        body,
        grid=_GRID,
        in_specs=[pl.BlockSpec((NC, _BLK, D), lambda i: (0, i, 0)),
                  pl.BlockSpec((NC, _BLK, D), lambda i: (0, i, 0)),
                  pl.BlockSpec((_BLK, DEGW), lambda i: (i, 0)),
                  pl.BlockSpec((D, D), lambda i: (0, 0)),
                  pl.BlockSpec((1, D), lambda i: (0, 0))],
        out_specs=pl.BlockSpec((NC, _BLK, D), lambda i: (0, i, 0)),
        out_shape=jax.ShapeDtypeStruct((NC, NPAD, D), jnp.float32),
    )(sp, u1, dinv16, w2, b1)


def _tc_final(sp, u2, dinv16, b2):
    def body(sp_ref, u_ref, dinv_ref, b_ref, o_ref):
        dinv = dinv_ref[:, 0:1]
        o_ref[...] = dinv * (sp_ref[0] + sp_ref[1] + u_ref[0]) + b_ref[...]

    return pl.pallas_call(
        body,
        grid=_GRID,
        in_specs=[pl.BlockSpec((NC, _BLK, D), lambda i: (0, i, 0)),
                  pl.BlockSpec((NC, _BLK, D), lambda i: (0, i, 0)),
                  pl.BlockSpec((_BLK, DEGW), lambda i: (i, 0)),
                  pl.BlockSpec((1, D), lambda i: (0, 0))],
        out_specs=pl.BlockSpec((_BLK, D), lambda i: (i, 0)),
        out_shape=jax.ShapeDtypeStruct((NPAD, D), jnp.float32),
    )(sp, u2, dinv16, b2)


def kernel(x, edge_index, W1, b1, W2, b2):
    src = edge_index[0]
    dst = edge_index[1]
    pad = jnp.full((EPAD - E,), N, jnp.int32)
    src_r = jnp.concatenate([src, pad]).reshape(TOTC, CHUNK)
    dst_r = jnp.concatenate([dst, pad]).reshape(TOTC, CHUNK)
    x_pad = jnp.concatenate([x, jnp.zeros((NPAD - N, D), x.dtype)], axis=0)
    b1r = b1.reshape(1, D)
    b2r = b2.reshape(1, D)

    degp = _sc_degree(dst_r)
    u1, dinv16 = _tc_first(x_pad, W1, degp)

    s1 = _sc_propagate(u1, src_r, dst_r)
    u2 = _tc_mid(s1, u1, dinv16, W2, b1r)

    s2 = _sc_propagate(u2, src_r, dst_r)
    z = _tc_final(s2, u2, dinv16, b2r)

    return z[:N]


# speedup vs baseline: 1.4081x; 1.4081x over previous
import functools

import jax
import jax.numpy as jnp
from jax import lax
from jax.experimental import pallas as pl
from jax.experimental.pallas import tpu as pltpu
from jax.experimental.pallas import tpu_sc as plsc

N = 10000
E = 320000
D = 128
NC = 2
NS = 16
NW = NC * NS
NPAD = 10240
RPS = NPAD // NS
CHUNK = 64
IBLK = 16
NBUF = 4
KTOT = 320
TOTC = NS * KTOT
EPAD = TOTC * CHUNK
NCHUNK = TOTC // NW
K0 = 304
K1 = 16
assert K0 + K1 == KTOT and K0 % IBLK == 0 and K1 % IBLK == 0
DEGW = 128

_HIGH = lax.Precision.HIGHEST


def _sc_mesh():
    return plsc.VectorSubcoreMesh(core_axis_name="c", subcore_axis_name="s")


def _sc_degree(dst_idx):
    @functools.partial(
        pl.kernel,
        out_type=jax.ShapeDtypeStruct((NC, NPAD, DEGW), jnp.float32),
        mesh=_sc_mesh(),
        scratch_types=[
            pltpu.VMEM((NCHUNK, CHUNK), jnp.int32),
            pltpu.VMEM((CHUNK, DEGW), jnp.float32),
            pltpu.VMEM((CHUNK, DEGW), jnp.float32),
            pltpu.VMEM_SHARED((NPAD, DEGW), jnp.float32),
            pltpu.SemaphoreType.DMA,
        ],
    )
    def k(d_hbm, out_hbm, didx, zbuf, ones, acc, sem):
        c = lax.axis_index("c")
        s = lax.axis_index("s")
        w = c * NS + s
        pltpu.sync_copy(d_hbm.at[pl.ds(w * NCHUNK, NCHUNK)], didx)

        @pl.loop(0, CHUNK)
        def _(i):
            zbuf.at[i, pl.ds(0, DEGW)][...] = jnp.zeros((DEGW,), jnp.float32)
            ones.at[i, pl.ds(0, DEGW)][...] = jnp.full((DEGW,), 1.0, jnp.float32)

        @pl.loop(0, RPS, step=CHUNK)
        def _(r):
            pltpu.sync_copy(zbuf, acc.at[pl.ds(s * RPS + r, CHUNK)])

        plsc.subcore_barrier()

        @pl.loop(0, NCHUNK, step=8)
        def _(j):
            for t in range(8):
                pltpu.async_copy(ones, acc.at[didx.at[j + t]], sem, add=True)
            for t in range(8):
                pltpu.make_async_copy(ones, acc.at[didx.at[j + t]], sem).wait()

        plsc.subcore_barrier()
        pltpu.sync_copy(acc.at[pl.ds(s * RPS, RPS)],
                        out_hbm.at[c, pl.ds(s * RPS, RPS)])

    return k(dst_idx)


def _sc_propagate(u_pad, src_idx, dst_idx):
    @functools.partial(
        pl.kernel,
        out_type=jax.ShapeDtypeStruct((NC, NPAD, D), jnp.float32),
        mesh=_sc_mesh(),
        scratch_types=[
            pltpu.VMEM((IBLK, CHUNK), jnp.int32),
            pltpu.VMEM((IBLK, CHUNK), jnp.int32),
        ] + [pltpu.VMEM((CHUNK, D), jnp.float32) for _ in range(NBUF)] + [
            pltpu.VMEM_SHARED((NPAD, D), jnp.float32),
        ] + [pltpu.SemaphoreType.DMA for _ in range(NBUF)],
    )
    def k(u_hbm, s_hbm, d_hbm, out_hbm, sidx, didx, *rest):
        rows = rest[:NBUF]
        acc = rest[NBUF]
        sems = rest[NBUF + 1:]
        c = lax.axis_index("c")
        s = lax.axis_index("s")
        u_mine = u_hbm.at[c]
        base = jnp.where(c == 0, s * K0, NS * K0 + s * K1)
        count = jnp.where(c == 0, K0, K1)

        @pl.loop(0, CHUNK)
        def _(i):
            @pl.loop(0, D, step=16)
            def _(j):
                rows[0].at[i, pl.ds(j, 16)][...] = jnp.zeros((16,), jnp.float32)

        @pl.loop(0, RPS, step=CHUNK)
        def _(r):
            pltpu.sync_copy(rows[0], acc.at[pl.ds(s * RPS + r, CHUNK)])

        plsc.subcore_barrier()

        def gather(j, q):
            return pltpu.async_copy(u_mine.at[sidx.at[j]], rows[q], sems[q])

        def gather_wait(j, q):
            pltpu.make_async_copy(u_mine.at[sidx.at[j]], rows[q], sems[q]).wait()

        @pl.loop(0, count, step=IBLK)
        def _(b):
            pltpu.sync_copy(s_hbm.at[pl.ds(base + b, IBLK)], sidx)
            pltpu.sync_copy(d_hbm.at[pl.ds(base + b, IBLK)], didx)
            for q in range(NBUF):
                gather(q, q)

            @pl.loop(0, IBLK, step=NBUF)
            def _(j):
                for q in range(NBUF):
                    gather_wait(j + q, q)
                    pltpu.sync_copy(rows[q], acc.at[didx.at[j + q]], add=True)

                    @pl.when(j + q + NBUF < IBLK)
                    def _():
                        gather(j + q + NBUF, q)

        plsc.subcore_barrier()
        pltpu.sync_copy(acc.at[pl.ds(s * RPS, RPS)],
                        out_hbm.at[c, pl.ds(s * RPS, RPS)])

    return k(u_pad, src_idx, dst_idx)


_BLK = 512
_GRID = (NPAD // _BLK,)


def _tc_first(x, w, degp):
    def body(x_ref, w_ref, deg_ref, u_ref, dinv_ref):
        deg = deg_ref[0][:, 0:1] + deg_ref[1][:, 0:1] + 1.0
        dinv = lax.rsqrt(deg)
        xw = jnp.dot(x_ref[...], w_ref[...], precision=_HIGH,
                     preferred_element_type=jnp.float32)
        u = xw * dinv
        u_ref[0] = u
        u_ref[1] = u
        dinv_ref[...] = jnp.broadcast_to(dinv, (_BLK, DEGW))

    return pl.pallas_call(
        body,
        grid=_GRID,
        in_specs=[pl.BlockSpec((_BLK, D), lambda i: (i, 0)),
                  pl.BlockSpec((D, D), lambda i: (0, 0)),
                  pl.BlockSpec((NC, _BLK, DEGW), lambda i: (0, i, 0))],
        out_specs=[pl.BlockSpec((NC, _BLK, D), lambda i: (0, i, 0)),
                   pl.BlockSpec((_BLK, DEGW), lambda i: (i, 0))],
        out_shape=[jax.ShapeDtypeStruct((NC, NPAD, D), jnp.float32),
                   jax.ShapeDtypeStruct((NPAD, DEGW), jnp.float32)],
    )(x, w, degp)


def _tc_mid(sp, u1, dinv16, w2, b1):
    def body(sp_ref, u_ref, dinv_ref, w_ref, b_ref, o_ref):
        dinv = dinv_ref[:, 0:1]
        pre = dinv * (sp_ref[0] + sp_ref[1] + u_ref[0]) + b_ref[...]
        h = jnp.maximum(pre, 0.0)
        u2 = jnp.dot(h, w_ref[...], precision=_HIGH,
                     preferred_element_type=jnp.float32) * dinv
        o_ref[0] = u2
        o_ref[1] = u2

    return pl.pallas_call(
        body,
        grid=_GRID,
        in_specs=[pl.BlockSpec((NC, _BLK, D), lambda i: (0, i, 0)),
                  pl.BlockSpec((NC, _BLK, D), lambda i: (0, i, 0)),
                  pl.BlockSpec((_BLK, DEGW), lambda i: (i, 0)),
                  pl.BlockSpec((D, D), lambda i: (0, 0)),
                  pl.BlockSpec((1, D), lambda i: (0, 0))],
        out_specs=pl.BlockSpec((NC, _BLK, D), lambda i: (0, i, 0)),
        out_shape=jax.ShapeDtypeStruct((NC, NPAD, D), jnp.float32),
    )(sp, u1, dinv16, w2, b1)


def _tc_final(sp, u2, dinv16, b2):
    def body(sp_ref, u_ref, dinv_ref, b_ref, o_ref):
        dinv = dinv_ref[:, 0:1]
        o_ref[...] = dinv * (sp_ref[0] + sp_ref[1] + u_ref[0]) + b_ref[...]

    return pl.pallas_call(
        body,
        grid=_GRID,
        in_specs=[pl.BlockSpec((NC, _BLK, D), lambda i: (0, i, 0)),
                  pl.BlockSpec((NC, _BLK, D), lambda i: (0, i, 0)),
                  pl.BlockSpec((_BLK, DEGW), lambda i: (i, 0)),
                  pl.BlockSpec((1, D), lambda i: (0, 0))],
        out_specs=pl.BlockSpec((_BLK, D), lambda i: (i, 0)),
        out_shape=jax.ShapeDtypeStruct((NPAD, D), jnp.float32),
    )(sp, u2, dinv16, b2)


def kernel(x, edge_index, W1, b1, W2, b2):
    src = edge_index[0]
    dst = edge_index[1]
    pad = jnp.full((EPAD - E,), N, jnp.int32)
    src_r = jnp.concatenate([src, pad]).reshape(TOTC, CHUNK)
    dst_r = jnp.concatenate([dst, pad]).reshape(TOTC, CHUNK)
    x_pad = jnp.concatenate([x, jnp.zeros((NPAD - N, D), x.dtype)], axis=0)
    b1r = b1.reshape(1, D)
    b2r = b2.reshape(1, D)

    degp = _sc_degree(dst_r)
    u1, dinv16 = _tc_first(x_pad, W1, degp)

    s1 = _sc_propagate(u1, src_r, dst_r)
    u2 = _tc_mid(s1, u1, dinv16, W2, b1r)

    s2 = _sc_propagate(u2, src_r, dst_r)
    z = _tc_final(s2, u2, dinv16, b2r)

    return z[:N]
